# Initial kernel scaffold; baseline (speedup 1.0000x reference)
import jax
import jax.numpy as jnp
from jax.experimental import pallas as pl


def kernel(index_list):
    raise NotImplementedError("write your pallas kernel here")


# speedup vs baseline: 1.5945x; 1.5945x over previous
import jax
import jax.numpy as jnp
from jax.experimental import pallas as pl

N_FIELDS = 26
N_EMB = 100
BATCH = 16384
OUT_W = N_FIELDS * N_EMB
BLOCK_ROWS = 256


def _onehot_block(idx_ref, out_ref):
    idx = idx_ref[...]
    for i in range(N_FIELDS):
        col = jax.lax.broadcasted_iota(jnp.int32, (BLOCK_ROWS, N_EMB), 1)
        field = idx[:, i][:, None]
        out_ref[:, i * N_EMB:(i + 1) * N_EMB] = (col == field).astype(jnp.int32)


def kernel(index_list):
    grid = (BATCH // BLOCK_ROWS,)
    return pl.pallas_call(
        _onehot_block,
        grid=grid,
        in_specs=[pl.BlockSpec((BLOCK_ROWS, N_FIELDS), lambda r: (r, 0))],
        out_specs=pl.BlockSpec((BLOCK_ROWS, OUT_W), lambda r: (r, 0)),
        out_shape=jax.ShapeDtypeStruct((BATCH, OUT_W), jnp.int32),
    )(index_list)


# speedup vs baseline: 2.2258x; 1.3960x over previous
import jax
import jax.numpy as jnp
from jax.experimental import pallas as pl

N_FIELDS = 26
N_EMB = 100
BATCH = 16384
OUT_W = N_FIELDS * N_EMB
BLOCK_ROWS = 256


def _onehot_block(idx_ref, sel_ref, mod_ref, out_ref):
    idx = idx_ref[...].astype(jnp.float32)
    expanded = jax.lax.dot_general(
        idx, sel_ref[...],
        dimension_numbers=(((1,), (0,)), ((), ())),
        preferred_element_type=jnp.float32,
    )
    out_ref[...] = (expanded == mod_ref[...]).astype(jnp.int32)


def kernel(index_list):
    sel = (jax.lax.broadcasted_iota(jnp.int32, (N_FIELDS, OUT_W), 0)
           == jax.lax.broadcasted_iota(jnp.int32, (N_FIELDS, OUT_W), 1) // N_EMB
           ).astype(jnp.float32)
    mod = (jax.lax.broadcasted_iota(jnp.int32, (1, OUT_W), 1) % N_EMB
           ).astype(jnp.float32)
    grid = (BATCH // BLOCK_ROWS,)
    return pl.pallas_call(
        _onehot_block,
        grid=grid,
        in_specs=[
            pl.BlockSpec((BLOCK_ROWS, N_FIELDS), lambda r: (r, 0)),
            pl.BlockSpec((N_FIELDS, OUT_W), lambda r: (0, 0)),
            pl.BlockSpec((1, OUT_W), lambda r: (0, 0)),
        ],
        out_specs=pl.BlockSpec((BLOCK_ROWS, OUT_W), lambda r: (r, 0)),
        out_shape=jax.ShapeDtypeStruct((BATCH, OUT_W), jnp.int32),
    )(index_list, sel, mod)
